# baseline (speedup 1.0000x reference)
import functools

import jax
import jax.numpy as jnp
from jax import lax
from jax.experimental import pallas as pl
from jax.experimental.pallas import tpu as pltpu
from jax.experimental.pallas import tpu_sc as plsc

H = 4096
W = 4096
N = H * W

NC = 2
NS = 16
L = 16
NW = NC * NS

SC_ROWS = 1536
ROWS_W = SC_ROWS // NW
PER_W = ROWS_W * W
CROWS = 8
CHUNK = CROWS * W
NCHUNK = ROWS_W // CROWS
U = 8
UNROLL = 4
GRP = U * L
GPR = W // GRP

BR = 256
TC_BLK0 = SC_ROWS // BR
NBLK = (H - SC_ROWS) // BR


def _mesh():
    return plsc.VectorSubcoreMesh(
        core_axis_name="c", subcore_axis_name="s", num_cores=NC, num_subcores=NS
    )


def _stage1_body(x_hbm, min_hbm, idx_hbm, buf0, buf1, buf2, mout, iout,
                 sem0, sem1, sem2):
    wid = lax.axis_index("s") * NC + lax.axis_index("c")
    base = wid * PER_W
    wrow = wid * ROWS_W
    la = lax.iota(jnp.int32, L)

    bufs = (buf0, buf1, buf2)
    sems = (sem0, sem1, sem2)
    NB = 3

    def copy(c):
        return pltpu.make_async_copy(
            x_hbm.at[pl.ds(wrow + c * CROWS, CROWS)], bufs[c % NB], sems[c % NB]
        )

    copy(0).start()
    copy(1).start()

    ms = tuple(jnp.full((L,), jnp.inf, jnp.float32) for _ in range(U))
    gis = tuple(jnp.zeros((L,), jnp.int32) for _ in range(U))
    g = jnp.zeros((L,), jnp.int32)

    for c in range(NCHUNK):
        if c + 2 < NCHUNK:
            copy(c + 2).start()
        copy(c).wait()
        buf = bufs[c % 3]

        for rr in range(CROWS):
            @plsc.parallel_loop(0, GPR, unroll=UNROLL, carry=(g, ms, gis))
            def gbody(i, carry, buf=buf, rr=rr):
                gg, mm, ggi = carry
                nm, ngi = [], []
                for u in range(U):
                    v = buf[rr, pl.ds(i * GRP + u * L, L)]
                    mask = v < mm[u]
                    nm.append(jnp.where(mask, v, mm[u]))
                    ngi.append(jnp.where(mask, gg, ggi[u]))
                return gg + 1, tuple(nm), tuple(ngi)

            g, ms, gis = gbody

    def merge(a, b):
        (ma, ia), (mb, ib) = a, b
        mask = (mb < ma) | ((mb == ma) & (ib < ia))
        return jnp.where(mask, mb, ma), jnp.where(mask, ib, ia)

    pairs = [(ms[u], base + gis[u] * GRP + (u * L) + la) for u in range(U)]
    while len(pairs) > 1:
        pairs = [merge(pairs[j], pairs[j + 1]) for j in range(0, len(pairs), 2)]
    m, mi = pairs[0]

    mout[...] = m
    iout[...] = mi
    pltpu.sync_copy(mout, min_hbm.at[wid])
    pltpu.sync_copy(iout, idx_hbm.at[wid])


def _tcscan_body(x_ref, val_ref, idx_ref, bestv, besti):
    i = pl.program_id(0)

    @pl.when(i == 0)
    def _():
        bestv[0] = jnp.inf
        besti[0] = jnp.int32(0)

    xb = x_ref[...]
    bmin = jnp.min(xb)

    @pl.when(bmin < bestv[0])
    def _():
        ri = lax.broadcasted_iota(jnp.int32, (BR, W), 0)
        ci = lax.broadcasted_iota(jnp.int32, (BR, W), 1)
        flat = ri * W + ci
        lidx = jnp.min(jnp.where(xb == bmin, flat, jnp.int32(2**31 - 1)))
        bestv[0] = bmin
        besti[0] = (SC_ROWS + i * BR) * W + lidx

    @pl.when(i == pl.num_programs(0) - 1)
    def _():
        val_ref[0] = bestv[0]
        idx_ref[0] = besti[0]


def _merge_body(pmin_ref, pidx_ref, tcv_ref, tci_ref, coords_hbm, out_ref,
                g0, g1, sem0, sem1):
    m = pmin_ref[...]
    mi = pidx_ref[...]
    gmin = jnp.min(m)
    cand = jnp.where(m == gmin, mi, jnp.int32(2**31 - 1))
    gidx = jnp.min(cand)

    tcv = tcv_ref[0]
    tci = tci_ref[0]
    tc_better = (tcv < gmin) | ((tcv == gmin) & (tci < gidx))
    gidx = jnp.where(tc_better, tci, gidx)

    gr = gidx >> 12
    gc = gidx & (W - 1)
    r0 = pl.multiple_of(gr & ~jnp.int32(7), 8)
    c0 = pl.multiple_of(gc & ~jnp.int32(127), 128)
    offr = gr & 7
    offc = gc & 127
    cp0 = pltpu.make_async_copy(
        coords_hbm.at[0, pl.ds(r0, 8), pl.ds(c0, 128)], g0, sem0
    )
    cp1 = pltpu.make_async_copy(
        coords_hbm.at[1, pl.ds(r0, 8), pl.ds(c0, 128)], g1, sem1
    )
    cp0.start()
    cp1.start()
    cp0.wait()
    cp1.wait()
    ri = lax.broadcasted_iota(jnp.int32, (8, 128), 0)
    ci = lax.broadcasted_iota(jnp.int32, (8, 128), 1)
    sel = (ri == offr) & (ci == offc)
    v0 = jnp.sum(jnp.where(sel, g0[...], 0.0))
    v1 = jnp.sum(jnp.where(sel, g1[...], 0.0))
    out_ref[0] = v0
    out_ref[1] = v1


@functools.cache
def _build():
    stage1 = pl.kernel(
        _stage1_body,
        out_type=(
            jax.ShapeDtypeStruct((NW, L), jnp.float32),
            jax.ShapeDtypeStruct((NW, L), jnp.int32),
        ),
        mesh=_mesh(),
        scratch_types=[
            pltpu.VMEM((CROWS, W), jnp.float32),
            pltpu.VMEM((CROWS, W), jnp.float32),
            pltpu.VMEM((CROWS, W), jnp.float32),
            pltpu.VMEM((L,), jnp.float32),
            pltpu.VMEM((L,), jnp.int32),
            pltpu.SemaphoreType.DMA,
            pltpu.SemaphoreType.DMA,
            pltpu.SemaphoreType.DMA,
        ],
    )
    tcscan = pl.pallas_call(
        _tcscan_body,
        grid=(NBLK,),
        out_shape=(
            jax.ShapeDtypeStruct((1,), jnp.float32),
            jax.ShapeDtypeStruct((1,), jnp.int32),
        ),
        in_specs=[pl.BlockSpec((BR, W), lambda i: (TC_BLK0 + i, 0))],
        out_specs=(
            pl.BlockSpec(memory_space=pltpu.SMEM),
            pl.BlockSpec(memory_space=pltpu.SMEM),
        ),
        scratch_shapes=[
            pltpu.SMEM((1,), jnp.float32),
            pltpu.SMEM((1,), jnp.int32),
        ],
    )
    merge = pl.pallas_call(
        _merge_body,
        out_shape=jax.ShapeDtypeStruct((2,), jnp.float32),
        in_specs=[
            pl.BlockSpec(memory_space=pltpu.VMEM),
            pl.BlockSpec(memory_space=pltpu.VMEM),
            pl.BlockSpec(memory_space=pltpu.SMEM),
            pl.BlockSpec(memory_space=pltpu.SMEM),
            pl.BlockSpec(memory_space=pl.ANY),
        ],
        out_specs=pl.BlockSpec(memory_space=pltpu.SMEM),
        scratch_shapes=[
            pltpu.VMEM((8, 128), jnp.float32),
            pltpu.VMEM((8, 128), jnp.float32),
            pltpu.SemaphoreType.DMA,
            pltpu.SemaphoreType.DMA,
        ],
    )
    return stage1, tcscan, merge


def kernel(x, coords2d):
    stage1, tcscan, merge = _build()
    x2 = x.reshape(H, W)
    pmin, pidx = stage1(x2)
    tcv, tci = tcscan(x2)
    return merge(pmin, pidx, tcv, tci, coords2d)


# speedup vs baseline: 1.0387x; 1.0387x over previous
import functools

import jax
import jax.numpy as jnp
from jax import lax
from jax.experimental import pallas as pl
from jax.experimental.pallas import tpu as pltpu
from jax.experimental.pallas import tpu_sc as plsc

H = 4096
W = 4096
N = H * W

NC = 2
NS = 16
L = 16
NW = NC * NS

SC_ROWS = 1536
ROWS_W = SC_ROWS // NW
PER_W = ROWS_W * W
CROWS = 8
CHUNK = CROWS * W
NCHUNK = ROWS_W // CROWS
U = 8
UNROLL = 4
GRP = U * L
GPR = W // GRP

BR = 256
TC_BLK0 = SC_ROWS // BR
NBLK = (H - SC_ROWS) // BR


def _mesh():
    return plsc.VectorSubcoreMesh(
        core_axis_name="c", subcore_axis_name="s", num_cores=NC, num_subcores=NS
    )


def _stage1_body(x_hbm, min_hbm, idx_hbm, buf0, buf1, mout, iout, sem0, sem1):
    wid = lax.axis_index("s") * NC + lax.axis_index("c")
    base = wid * PER_W
    wrow = wid * ROWS_W
    la = lax.iota(jnp.int32, L)

    bufs = (buf0, buf1)
    sems = (sem0, sem1)
    NB = 2

    def copy(c):
        return pltpu.make_async_copy(
            x_hbm.at[pl.ds(wrow + c * CROWS, CROWS)], bufs[c % NB], sems[c % NB]
        )

    copy(0).start()

    ms = tuple(jnp.full((L,), jnp.inf, jnp.float32) for _ in range(U))
    gis = tuple(jnp.zeros((L,), jnp.int32) for _ in range(U))
    g = jnp.zeros((L,), jnp.int32)

    for c in range(NCHUNK):
        if c + 1 < NCHUNK:
            copy(c + 1).start()
        copy(c).wait()
        buf = bufs[c % 2]

        for rr in range(CROWS):
            @plsc.parallel_loop(0, GPR, unroll=UNROLL, carry=(g, ms, gis))
            def gbody(i, carry, buf=buf, rr=rr):
                gg, mm, ggi = carry
                nm, ngi = [], []
                for u in range(U):
                    v = buf[rr, pl.ds(i * GRP + u * L, L)]
                    mask = v < mm[u]
                    nm.append(jnp.where(mask, v, mm[u]))
                    ngi.append(jnp.where(mask, gg, ggi[u]))
                return gg + 1, tuple(nm), tuple(ngi)

            g, ms, gis = gbody

    def merge(a, b):
        (ma, ia), (mb, ib) = a, b
        mask = (mb < ma) | ((mb == ma) & (ib < ia))
        return jnp.where(mask, mb, ma), jnp.where(mask, ib, ia)

    pairs = [(ms[u], base + gis[u] * GRP + (u * L) + la) for u in range(U)]
    while len(pairs) > 1:
        pairs = [merge(pairs[j], pairs[j + 1]) for j in range(0, len(pairs), 2)]
    m, mi = pairs[0]

    mout[...] = m
    iout[...] = mi
    pltpu.sync_copy(mout, min_hbm.at[wid])
    pltpu.sync_copy(iout, idx_hbm.at[wid])


def _tcscan_body(x_ref, val_ref, idx_ref, bestv, besti):
    i = pl.program_id(0)

    @pl.when(i == 0)
    def _():
        bestv[0] = jnp.inf
        besti[0] = jnp.int32(0)

    xb = x_ref[...]
    bmin = jnp.min(xb)

    @pl.when(bmin < bestv[0])
    def _():
        ri = lax.broadcasted_iota(jnp.int32, (BR, W), 0)
        ci = lax.broadcasted_iota(jnp.int32, (BR, W), 1)
        flat = ri * W + ci
        lidx = jnp.min(jnp.where(xb == bmin, flat, jnp.int32(2**31 - 1)))
        bestv[0] = bmin
        besti[0] = (SC_ROWS + i * BR) * W + lidx

    @pl.when(i == pl.num_programs(0) - 1)
    def _():
        val_ref[0] = bestv[0]
        idx_ref[0] = besti[0]


def _merge_body(pmin_ref, pidx_ref, tcv_ref, tci_ref, coords_hbm, out_ref,
                g0, g1, sem0, sem1):
    m = pmin_ref[...]
    mi = pidx_ref[...]
    gmin = jnp.min(m)
    cand = jnp.where(m == gmin, mi, jnp.int32(2**31 - 1))
    gidx = jnp.min(cand)

    tcv = tcv_ref[0]
    tci = tci_ref[0]
    tc_better = (tcv < gmin) | ((tcv == gmin) & (tci < gidx))
    gidx = jnp.where(tc_better, tci, gidx)

    gr = gidx >> 12
    gc = gidx & (W - 1)
    r0 = pl.multiple_of(gr & ~jnp.int32(7), 8)
    c0 = pl.multiple_of(gc & ~jnp.int32(127), 128)
    offr = gr & 7
    offc = gc & 127
    cp0 = pltpu.make_async_copy(
        coords_hbm.at[0, pl.ds(r0, 8), pl.ds(c0, 128)], g0, sem0
    )
    cp1 = pltpu.make_async_copy(
        coords_hbm.at[1, pl.ds(r0, 8), pl.ds(c0, 128)], g1, sem1
    )
    cp0.start()
    cp1.start()
    cp0.wait()
    cp1.wait()
    ri = lax.broadcasted_iota(jnp.int32, (8, 128), 0)
    ci = lax.broadcasted_iota(jnp.int32, (8, 128), 1)
    sel = (ri == offr) & (ci == offc)
    v0 = jnp.sum(jnp.where(sel, g0[...], 0.0))
    v1 = jnp.sum(jnp.where(sel, g1[...], 0.0))
    out_ref[0] = v0
    out_ref[1] = v1


@functools.cache
def _build():
    stage1 = pl.kernel(
        _stage1_body,
        out_type=(
            jax.ShapeDtypeStruct((NW, L), jnp.float32),
            jax.ShapeDtypeStruct((NW, L), jnp.int32),
        ),
        mesh=_mesh(),
        scratch_types=[
            pltpu.VMEM((CROWS, W), jnp.float32),
            pltpu.VMEM((CROWS, W), jnp.float32),
            pltpu.VMEM((L,), jnp.float32),
            pltpu.VMEM((L,), jnp.int32),
            pltpu.SemaphoreType.DMA,
            pltpu.SemaphoreType.DMA,
        ],
    )
    tcscan = pl.pallas_call(
        _tcscan_body,
        grid=(NBLK,),
        out_shape=(
            jax.ShapeDtypeStruct((1,), jnp.float32),
            jax.ShapeDtypeStruct((1,), jnp.int32),
        ),
        in_specs=[pl.BlockSpec((BR, W), lambda i: (TC_BLK0 + i, 0))],
        out_specs=(
            pl.BlockSpec(memory_space=pltpu.SMEM),
            pl.BlockSpec(memory_space=pltpu.SMEM),
        ),
        scratch_shapes=[
            pltpu.SMEM((1,), jnp.float32),
            pltpu.SMEM((1,), jnp.int32),
        ],
    )
    merge = pl.pallas_call(
        _merge_body,
        out_shape=jax.ShapeDtypeStruct((2,), jnp.float32),
        in_specs=[
            pl.BlockSpec(memory_space=pltpu.VMEM),
            pl.BlockSpec(memory_space=pltpu.VMEM),
            pl.BlockSpec(memory_space=pltpu.SMEM),
            pl.BlockSpec(memory_space=pltpu.SMEM),
            pl.BlockSpec(memory_space=pl.ANY),
        ],
        out_specs=pl.BlockSpec(memory_space=pltpu.SMEM),
        scratch_shapes=[
            pltpu.VMEM((8, 128), jnp.float32),
            pltpu.VMEM((8, 128), jnp.float32),
            pltpu.SemaphoreType.DMA,
            pltpu.SemaphoreType.DMA,
        ],
    )
    return stage1, tcscan, merge


def kernel(x, coords2d):
    stage1, tcscan, merge = _build()
    x2 = x.reshape(H, W)
    pmin, pidx = stage1(x2)
    tcv, tci = tcscan(x2)
    return merge(pmin, pidx, tcv, tci, coords2d)
